# Initial kernel scaffold; baseline (speedup 1.0000x reference)
import jax
import jax.numpy as jnp
from jax.experimental import pallas as pl


def kernel(task_seq, status_seq, emb_table, gru_Wih, gru_Whh, gru_bih, gru_bhh, pred_W, pred_b):
    raise NotImplementedError("write your pallas kernel here")


# speedup vs baseline: 11.4336x; 11.4336x over previous
import jax
import jax.numpy as jnp
from jax.experimental import pallas as pl
from jax.experimental.pallas import tpu as pltpu

_NT = 27
_H = 128
_SEQ = 20
_NE = _NT * 3


def _gkt_kernel(taskc_ref, idx3c_ref, taskt_ref, emb_ref, wet_ref, wat_ref,
                whht_ref, bih_ref, bhh_ref, pw_ref, pb_ref, outs_ref, hid_ref):
    B = taskc_ref.shape[0]
    f32 = jnp.float32

    gi_tab = jnp.dot(emb_ref[...], wet_ref[...],
                     preferred_element_type=f32) + bih_ref[...]
    wat = wat_ref[...]
    whht = whht_ref[...]
    bhh = bhh_ref[...]
    pw = pw_ref[...]
    pb = pb_ref[0, 0]

    iota81 = jax.lax.broadcasted_iota(jnp.int32, (B, _NE), 1)
    iota27l = jax.lax.broadcasted_iota(jnp.int32, (_NT, B), 0)

    zero_plane = jnp.zeros((B, _H), f32)
    for n in range(_NT):
        hid_ref[n] = zero_plane

    S = jnp.zeros((B, _H), f32)
    dT = jnp.full((_NT, B), pb, f32)
    prev_h = zero_plane
    inv28 = f32(1.0 / 28.0)

    col_masks = [taskc_ref[:, t:t + 1] for t in range(_SEQ)]

    for t in range(_SEQ):
        idx3c = idx3c_ref[:, t:t + 1]

        oh81 = (idx3c == iota81).astype(f32)
        gi_e = jnp.dot(oh81, gi_tab, preferred_element_type=f32)

        x = S + prev_h
        gi = gi_e + jnp.dot(x, wat, preferred_element_type=f32) * inv28
        gh = jnp.dot(prev_h, whht, preferred_element_type=f32) + bhh

        r = jax.nn.sigmoid(gi[:, :_H] + gh[:, :_H])
        z = jax.nn.sigmoid(gi[:, _H:2 * _H] + gh[:, _H:2 * _H])
        nn = jnp.tanh(gi[:, 2 * _H:] + r * gh[:, 2 * _H:])
        new_h = nn + z * (prev_h - nn)

        taskc = col_masks[t]
        next_h = zero_plane
        for n in range(_NT):
            old = hid_ref[n]
            upd = jnp.where(taskc == n, new_h, old)
            hid_ref[n] = upd
            if t + 1 < _SEQ:
                next_h = next_h + jnp.where(col_masks[t + 1] == n, upd, f32(0.0))

        S = S + new_h - prev_h
        prev_h = next_h

        lnewT = jax.lax.dot_general(pw, new_h, (((1,), (1,)), ((), ())),
                                    preferred_element_type=f32) + pb
        taskt = taskt_ref[t:t + 1, :]
        dT = jnp.where(iota27l == taskt, lnewT, dT)
        outs_ref[t] = dT


def kernel(task_seq, status_seq, emb_table, gru_Wih, gru_Whh, gru_bih,
           gru_bhh, pred_W, pred_b):
    B = task_seq.shape[0]
    f32 = jnp.float32

    idx3 = task_seq * 3 + status_seq
    taskT = jnp.transpose(task_seq)
    wet = jnp.transpose(gru_Wih[:, :_H])
    wat = jnp.transpose(gru_Wih[:, _H:])
    whht = jnp.transpose(gru_Whh)
    bih = gru_bih.reshape(1, 3 * _H).astype(f32)
    bhh = gru_bhh.reshape(1, 3 * _H).astype(f32)
    pw = pred_W.reshape(1, _H).astype(f32)
    pb = pred_b.reshape(1, 1).astype(f32)

    outs_raw, hid_raw = pl.pallas_call(
        _gkt_kernel,
        out_shape=[
            jax.ShapeDtypeStruct((_SEQ, _NT, B), f32),
            jax.ShapeDtypeStruct((_NT, B, _H), f32),
        ],
    )(task_seq, idx3, taskT, emb_table.astype(f32), wet, wat, whht,
      bih, bhh, pw, pb)

    outs = jnp.transpose(outs_raw, (2, 0, 1))
    hidden = jnp.transpose(hid_raw, (1, 0, 2))
    return outs, hidden
